# baseline (speedup 1.0000x reference)
import jax
import jax.numpy as jnp
from jax.experimental import pallas as pl

_N = 10000
_F_IN = 128
_F_H = 32


def _fused_kernel(x_ref, wz_ref, wh_ref, bz_ref, bh_ref, wlin_ref, blin_ref,
                  out_ref):
    wz = wz_ref[0, 0, :_F_IN, :] + wz_ref[1, 0, :_F_IN, :]
    wh = wh_ref[0, 0, :_F_IN, :] + wh_ref[1, 0, :_F_IN, :]
    w = jnp.concatenate([wz * 0.5, wh], axis=1)
    b = jnp.concatenate([bz_ref[...] * 0.5, bh_ref[...]], axis=1)
    y = jnp.dot(x_ref[...], w, preferred_element_type=jnp.float32) + b
    u = jnp.tanh(y)
    p = (1.0 - u[:, :_F_H]) * u[:, _F_H:]
    s = jnp.sum(jnp.maximum(p, 0.0) * wlin_ref[...], keepdims=True)
    out_ref[...] = s * (0.5 / _N) + blin_ref[...]


def kernel(x, edge_index, edge_weight, W_z, b_z, W_r, b_r, W_h, b_h,
           W_lin, b_lin):
    del edge_index, edge_weight, W_r, b_r
    return pl.pallas_call(
        _fused_kernel,
        out_shape=jax.ShapeDtypeStruct((1, 1), jnp.float32),
    )(x, W_z, W_h, b_z.reshape(1, _F_H), b_h.reshape(1, _F_H),
      W_lin, b_lin.reshape(1, 1))
